# Initial kernel scaffold; baseline (speedup 1.0000x reference)
import jax
import jax.numpy as jnp
from jax.experimental import pallas as pl


def kernel(x, var_idx, adjc, embeddings, W, b):
    raise NotImplementedError("write your pallas kernel here")


# speedup vs baseline: 2.6492x; 2.6492x over previous
import functools

import jax
import jax.numpy as jnp
from jax import lax
from jax.experimental import pallas as pl
from jax.experimental.pallas import tpu as pltpu
from jax.experimental.pallas import tpu_sc as plsc

N_NODES = 49152
F = 64
NVARS = 4
B = 2
V = 4

_NC = 2
_NS = 16
_NW = _NC * _NS

_R = NVARS * N_NODES
_RPW = _R // _NW
_IDX_PER_XFER = 128
_XFERS_PER_BLK = 8
_BLK_ROWS = _IDX_PER_XFER * _XFERS_PER_BLK
_NBLK = _RPW // _BLK_ROWS
_XPW = _RPW // _IDX_PER_XFER


def _sc_gather(table, idx3):
    mesh = plsc.VectorSubcoreMesh(core_axis_name="c", subcore_axis_name="s")

    @functools.partial(
        pl.kernel,
        out_type=jax.ShapeDtypeStruct((_R, F), jnp.float32),
        mesh=mesh,
        compiler_params=pltpu.CompilerParams(use_tc_tiling_on_sc=False),
        scratch_types=[
            pltpu.VMEM((_XPW, _IDX_PER_XFER), jnp.int32),
            pltpu.VMEM((_BLK_ROWS, F), jnp.float32),
            pltpu.SemaphoreType.DMA,
        ],
    )
    def gather_kernel(table_hbm, idx_hbm, out_hbm, idx_v, rows_v, sem):
        wid = lax.axis_index("s") * _NC + lax.axis_index("c")
        base = wid * _RPW
        pltpu.sync_copy(idx_hbm.at[wid], idx_v)

        def blk_body(blk, _):
            copies = []
            for j in range(_XFERS_PER_BLK):
                copies.append(pltpu.async_copy(
                    table_hbm.at[idx_v.at[blk * _XFERS_PER_BLK + j]],
                    rows_v.at[pl.ds(j * _IDX_PER_XFER, _IDX_PER_XFER)],
                    sem,
                ))
            for c in copies:
                c.wait()
            pltpu.sync_copy(
                rows_v,
                out_hbm.at[pl.ds(base + blk * _BLK_ROWS, _BLK_ROWS)],
            )
            return ()

        lax.fori_loop(0, _NBLK, blk_body, (), unroll=False)

    return gather_kernel(table, idx3)


_BN = 4096


def _film_body(vi_ref, x_ref, eg_ref, w_ref, b_ref, o_ref):
    m = jnp.dot(eg_ref[0], w_ref[...], preferred_element_type=jnp.float32)
    m = m + b_ref[...]
    scale = m[:, :F]
    shift = m[:, F:]
    o_ref[0, 0, 0] = x_ref[0, 0, 0] * scale + shift


def _tc_film(vi, x, eg, W, b2):
    grid = (B * V, N_NODES // _BN)
    grid_spec = pltpu.PrefetchScalarGridSpec(
        num_scalar_prefetch=1,
        grid=grid,
        in_specs=[
            pl.BlockSpec(
                (1, 1, 1, _BN, F),
                lambda bv, n, vi_ref: (bv // V, 0, bv % V, n, 0),
            ),
            pl.BlockSpec(
                (1, _BN, F),
                lambda bv, n, vi_ref: (vi_ref[bv], n, 0),
            ),
            pl.BlockSpec((F, 2 * F), lambda bv, n, vi_ref: (0, 0)),
            pl.BlockSpec((1, 2 * F), lambda bv, n, vi_ref: (0, 0)),
        ],
        out_specs=pl.BlockSpec(
            (1, 1, 1, _BN, F),
            lambda bv, n, vi_ref: (bv // V, 0, bv % V, n, 0),
        ),
    )
    return pl.pallas_call(
        _film_body,
        grid_spec=grid_spec,
        out_shape=jax.ShapeDtypeStruct(x.shape, x.dtype),
    )(vi, x, eg, W, b2)


def kernel(x, var_idx, adjc, embeddings, W, b):
    node_idx = adjc[:, 0].astype(jnp.int32)
    offs = (jnp.arange(NVARS, dtype=jnp.int32) * N_NODES)[:, None]
    idx3 = (offs + node_idx[None, :]).reshape(_NW, _XPW, _IDX_PER_XFER)
    table = embeddings.reshape(_R, F)
    eg = _sc_gather(table, idx3).reshape(NVARS, N_NODES, F)
    vi = var_idx.reshape(B * V).astype(jnp.int32)
    return _tc_film(vi, x, eg, W, b.reshape(1, 2 * F))
